# baseline (speedup 1.0000x reference)
import functools
import math

import jax
import jax.numpy as jnp
from jax import lax
from jax.experimental import pallas as pl
from jax.experimental.pallas import tpu as pltpu
from jax.experimental.pallas import tpu_sc as plsc

_B, _C = 1024, 100000
_S = 64.0
_M2 = 0.3
_M3 = 0.2
_CM2 = math.cos(_M2)
_SM2 = math.sin(_M2)

_NC, _NS, _L = 2, 16, 16
_NW = _NC * _NS
_RPW = _B // _NW
_CH = 20000
_NV = _CH // _L
_TS = (_RPW * _C) // _CH


def _margin_values(x):
    y = jnp.maximum(1.0 - x * x, 1e-12)
    i = lax.bitcast_convert_type(y, jnp.int32)
    r = lax.bitcast_convert_type(0x5F3759DF - (i >> 1), jnp.float32)
    for _ in range(3):
        r = r * (1.5 - 0.5 * y * r * r)
    sq = y * r
    return (x * _CM2 - sq * _SM2 - _M3) * _S


def _sc_body(flat_hbm, label_hbm, out_hbm, lab_v, idx_v, x_v, v_v,
             ibuf0, ibuf1, obuf0, obuf1, gsem, isem0, isem1, osem0, osem1,
             ssem):
    wid = lax.axis_index("s") * _NC + lax.axis_index("c")
    base = wid * _RPW
    fbase = base * _C

    pltpu.sync_copy(label_hbm.at[pl.ds(base, _RPW)], lab_v)
    for k in range(_RPW // _L):
        lab16 = jnp.maximum(lab_v[pl.ds(k * _L, _L)], 0)
        rows16 = (base + k * _L) + lax.iota(jnp.int32, _L)
        idx_v[pl.ds(k * _L, _L)] = rows16 * _C + lab16
    pltpu.async_copy(flat_hbm.at[idx_v], x_v, gsem).wait()
    for k in range(_RPW // _L):
        v_v[pl.ds(k * _L, _L)] = _margin_values(x_v[pl.ds(k * _L, _L)])

    ibufs = (ibuf0, ibuf1)
    obufs = (obuf0, obuf1)
    isems = (isem0, isem1)
    osems = (osem0, osem1)
    pltpu.async_copy(flat_hbm.at[pl.ds(fbase, _CH)], ibuf0, isem0)
    pltpu.async_copy(flat_hbm.at[pl.ds(fbase + _CH, _CH)], ibuf1, isem1)

    def step_slot(t, j):
        off = fbase + t * _CH
        pltpu.make_async_copy(
            flat_hbm.at[pl.ds(off, _CH)], ibufs[j], isems[j]).wait()

        @pl.when(t >= 2)
        def _():
            pltpu.make_async_copy(
                obufs[j], out_hbm.at[pl.ds(off, _CH)], osems[j]).wait()

        @plsc.parallel_loop(0, _NV, unroll=8)
        def _(i):
            obufs[j][pl.ds(i * _L, _L)] = ibufs[j][pl.ds(i * _L, _L)] * _S

        @pl.when(t + 2 < _TS)
        def _():
            pltpu.async_copy(
                flat_hbm.at[pl.ds(off + 2 * _CH, _CH)], ibufs[j], isems[j])

        pltpu.async_copy(obufs[j], out_hbm.at[pl.ds(off, _CH)], osems[j])

    @pl.loop(0, _TS, step=2)
    def _(t):
        step_slot(t, 0)
        step_slot(t + 1, 1)

    pltpu.make_async_copy(obuf0, out_hbm.at[pl.ds(fbase, _CH)], osem0).wait()
    pltpu.make_async_copy(obuf1, out_hbm.at[pl.ds(fbase, _CH)], osem1).wait()

    pltpu.async_copy(v_v, out_hbm.at[idx_v], ssem).wait()


@functools.cache
def _sc_combine():
    return pl.kernel(
        _sc_body,
        mesh=plsc.VectorSubcoreMesh(core_axis_name="c", subcore_axis_name="s"),
        out_type=jax.ShapeDtypeStruct((_B * _C,), jnp.float32),
        scratch_types=[
            pltpu.VMEM((_RPW,), jnp.int32),
            pltpu.VMEM((_RPW,), jnp.int32),
            pltpu.VMEM((_RPW,), jnp.float32),
            pltpu.VMEM((_RPW,), jnp.float32),
            pltpu.VMEM((_CH,), jnp.float32),
            pltpu.VMEM((_CH,), jnp.float32),
            pltpu.VMEM((_CH,), jnp.float32),
            pltpu.VMEM((_CH,), jnp.float32),
            pltpu.SemaphoreType.DMA,
            pltpu.SemaphoreType.DMA,
            pltpu.SemaphoreType.DMA,
            pltpu.SemaphoreType.DMA,
            pltpu.SemaphoreType.DMA,
            pltpu.SemaphoreType.DMA,
        ],
    )


def kernel(cosine, label):
    out = _sc_combine()(cosine.reshape(_B * _C), label)
    return out.reshape(_B, _C)
